# baseline (speedup 1.0000x reference)
import functools

import jax
import jax.numpy as jnp
from jax import lax
from jax.experimental import pallas as pl
from jax.experimental.pallas import tpu as pltpu
from jax.experimental.pallas import tpu_sc as plsc

N_TOK = 32768
D = 4096
BITS = 6

BM = 1024
M_SC = 8192
M_TC = N_TOK - M_SC

NW = 32
RPW = M_SC // NW
TB = 8
NB = 2
TBC = 4


def _lsh_tc_kernel(x_ref, w_ref, gates_ref, dec_ref):
    xb = x_ref[...].astype(jnp.bfloat16)
    wb = w_ref[...].astype(jnp.bfloat16)
    ht = jax.lax.dot_general(
        wb, xb, (((0,), (1,)), ((), ())),
        preferred_element_type=jnp.float32,
    )
    powers = (1 << jnp.arange(BITS - 1, -1, -1, dtype=jnp.int32)).astype(
        jnp.float32)[:, None]
    dec = jnp.sum(jnp.where(ht > 0, powers, 0.0), axis=0)
    dec_ref[...] = dec
    gates_ref[...] = jnp.ones_like(dec)


_GATHER_DNUMS = lax.GatherDimensionNumbers(
    offset_dims=(), collapsed_slice_dims=(0,), start_index_map=(0,))


def _shuffle(v, idx):
    return lax.gather(
        v, idx[:, None], _GATHER_DNUMS, slice_sizes=(1,),
        mode=lax.GatherScatterMode.PROMISE_IN_BOUNDS)


def _round_bf16(v):
    u = lax.bitcast_convert_type(v, jnp.uint32)
    rb = (u >> 16) & jnp.uint32(1)
    u = (u + jnp.uint32(0x7FFF) + rb) & jnp.uint32(0xFFFF0000)
    return lax.bitcast_convert_type(u, jnp.float32)


def _hsum(v, lane):
    for sh in (8, 4, 2, 1):
        v = v + _shuffle(v, lane ^ sh)
    return v


def _lsh_sc_body(x_hbm, wt_hbm, gates_hbm, dec_hbm,
                 wt_v, xbuf, decbuf, gbuf, sems):
    cid = lax.axis_index("c")
    sid = lax.axis_index("s")
    wid = sid * 2 + cid
    row0 = wid * RPW
    xrow0 = M_TC + row0

    pltpu.sync_copy(wt_hbm, wt_v)

    def round_w(ci, carry):
        base = ci * 16
        for j in range(BITS):
            wt_v[j, pl.ds(base, 16)] = _round_bf16(wt_v[j, pl.ds(base, 16)])
        return carry

    lax.fori_loop(0, D // 16, round_w, 0)

    ones16 = jnp.ones((16,), jnp.float32)
    for i in range(RPW // 16):
        gbuf[pl.ds(i * 16, 16)] = ones16

    def start_dma(step, b):
        pltpu.make_async_copy(
            x_hbm.at[pl.ds(xrow0 + step * TB, TB), :],
            xbuf.at[b],
            sems.at[b],
        ).start()

    def wait_dma(step, b):
        pltpu.make_async_copy(
            x_hbm.at[pl.ds(xrow0 + step * TB, TB), :],
            xbuf.at[b],
            sems.at[b],
        ).wait()

    for b in range(NB):
        start_dma(b, b)

    nsteps = RPW // TB

    lane = lax.iota(jnp.int32, 16)

    def outer(i, carry):
        g0 = i * NB
        decv = jnp.zeros((16,), jnp.float32)
        for b in range(NB):
            g = g0 + b
            wait_dma(g, b)

            for sub in range(TB // TBC):

                def chunk(ci, accs, _b=b, _sub=sub):
                    base = ci * 16
                    wvs = [wt_v[j, pl.ds(base, 16)] for j in range(BITS)]
                    out = []
                    for t in range(TBC):
                        xv = xbuf[_b, _sub * TBC + t, pl.ds(base, 16)]
                        xr = _round_bf16(xv)
                        for j in range(BITS):
                            out.append(accs[t * BITS + j] + xr * wvs[j])
                    return tuple(out)

                accs = lax.fori_loop(
                    0, D // 16, chunk,
                    tuple(jnp.zeros((16,), jnp.float32)
                          for _ in range(TBC * BITS)))

                for t in range(TBC):
                    dec = jnp.zeros((16,), jnp.float32)
                    for j in range(BITS):
                        h = _hsum(accs[t * BITS + j], lane)
                        dec = dec + jnp.where(
                            h > 0, jnp.float32(1 << (BITS - 1 - j)),
                            jnp.float32(0))
                    decv = jnp.where(
                        lane == b * TB + sub * TBC + t, dec, decv)

            @pl.when(g + NB < nsteps)
            def _(_g=g, _b=b):
                start_dma(_g + NB, _b)

        decbuf[pl.ds(i * (NB * TB), NB * TB)] = decv
        return carry

    lax.fori_loop(0, nsteps // NB, outer, 0)

    pltpu.sync_copy(decbuf, dec_hbm.at[pl.ds(row0, RPW)])
    pltpu.sync_copy(gbuf, gates_hbm.at[pl.ds(row0, RPW)])


def _tc_part(x, W):
    return pl.pallas_call(
        _lsh_tc_kernel,
        grid=(M_TC // BM,),
        in_specs=[
            pl.BlockSpec((BM, D), lambda i: (i, 0)),
            pl.BlockSpec((D, BITS), lambda i: (0, 0)),
        ],
        out_specs=[
            pl.BlockSpec((BM,), lambda i: (i,)),
            pl.BlockSpec((BM,), lambda i: (i,)),
        ],
        out_shape=[
            jax.ShapeDtypeStruct((M_TC,), jnp.float32),
            jax.ShapeDtypeStruct((M_TC,), jnp.float32),
        ],
    )(x, W)


def _sc_part(x, wt):
    mesh = plsc.VectorSubcoreMesh(core_axis_name="c", subcore_axis_name="s")
    run = pl.kernel(
        _lsh_sc_body,
        out_type=[
            jax.ShapeDtypeStruct((M_SC,), jnp.float32),
            jax.ShapeDtypeStruct((M_SC,), jnp.float32),
        ],
        mesh=mesh,
        scratch_types=[
            pltpu.VMEM((BITS, D), jnp.float32),
            pltpu.VMEM((NB, TB, D), jnp.float32),
            pltpu.VMEM((RPW,), jnp.float32),
            pltpu.VMEM((RPW,), jnp.float32),
            pltpu.SemaphoreType.DMA((NB,)),
        ],
    )
    return run(x, wt)


def _sc_part_interp(x, wt):
    mesh = plsc.VectorSubcoreMesh(core_axis_name="c", subcore_axis_name="s")
    run = pl.kernel(
        _lsh_sc_body,
        out_type=[
            jax.ShapeDtypeStruct((M_SC,), jnp.float32),
            jax.ShapeDtypeStruct((M_SC,), jnp.float32),
        ],
        mesh=mesh,
        interpret=True,
        scratch_types=[
            pltpu.VMEM((BITS, D), jnp.float32),
            pltpu.VMEM((NB, TB, D), jnp.float32),
            pltpu.VMEM((RPW,), jnp.float32),
            pltpu.VMEM((RPW,), jnp.float32),
            pltpu.SemaphoreType.DMA((NB,)),
        ],
    )
    return run(x, wt)


def kernel(x, W):
    wt = W.T
    gates_tc, dec_tc = _tc_part(x, W)
    gates_sc, dec_sc = _sc_part(x, wt)
    gates = jnp.concatenate([gates_tc, gates_sc])
    dec = jnp.concatenate([dec_tc, dec_sc])
    return gates, dec


# speedup vs baseline: 1.4356x; 1.4356x over previous
import functools

import jax
import jax.numpy as jnp
from jax import lax
from jax.experimental import pallas as pl
from jax.experimental.pallas import tpu as pltpu
from jax.experimental.pallas import tpu_sc as plsc

N_TOK = 32768
D = 4096
BITS = 6

BM = 1024
M_SC = 4096
M_TC = N_TOK - M_SC

NW = 32
RPW = M_SC // NW
TB = 8
NB = 2
TBC = 4


def _lsh_tc_kernel(x_ref, w_ref, gates_ref, dec_ref):
    xb = x_ref[...].astype(jnp.bfloat16)
    wb = w_ref[...].astype(jnp.bfloat16)
    ht = jax.lax.dot_general(
        wb, xb, (((0,), (1,)), ((), ())),
        preferred_element_type=jnp.float32,
    )
    powers = (1 << jnp.arange(BITS - 1, -1, -1, dtype=jnp.int32)).astype(
        jnp.float32)[:, None]
    dec = jnp.sum(jnp.where(ht > 0, powers, 0.0), axis=0)
    dec_ref[...] = dec
    gates_ref[...] = jnp.ones_like(dec)


_GATHER_DNUMS = lax.GatherDimensionNumbers(
    offset_dims=(), collapsed_slice_dims=(0,), start_index_map=(0,))


def _shuffle(v, idx):
    return lax.gather(
        v, idx[:, None], _GATHER_DNUMS, slice_sizes=(1,),
        mode=lax.GatherScatterMode.PROMISE_IN_BOUNDS)


def _round_bf16(v):
    u = lax.bitcast_convert_type(v, jnp.uint32)
    rb = (u >> 16) & jnp.uint32(1)
    u = (u + jnp.uint32(0x7FFF) + rb) & jnp.uint32(0xFFFF0000)
    return lax.bitcast_convert_type(u, jnp.float32)


def _hsum(v, lane):
    for sh in (8, 4, 2, 1):
        v = v + _shuffle(v, lane ^ sh)
    return v


def _lsh_sc_body(x_hbm, wt_hbm, gates_hbm, dec_hbm,
                 wt_v, xbuf, decbuf, gbuf, sems):
    cid = lax.axis_index("c")
    sid = lax.axis_index("s")
    wid = sid * 2 + cid
    row0 = wid * RPW
    xrow0 = M_TC + row0

    pltpu.sync_copy(wt_hbm, wt_v)

    def round_w(ci, carry):
        base = ci * 16
        for j in range(BITS):
            wt_v[j, pl.ds(base, 16)] = _round_bf16(wt_v[j, pl.ds(base, 16)])
        return carry

    lax.fori_loop(0, D // 16, round_w, 0)

    ones16 = jnp.ones((16,), jnp.float32)
    for i in range(RPW // 16):
        gbuf[pl.ds(i * 16, 16)] = ones16

    def start_dma(step, b):
        pltpu.make_async_copy(
            x_hbm.at[pl.ds(xrow0 + step * TB, TB), :],
            xbuf.at[b],
            sems.at[b],
        ).start()

    def wait_dma(step, b):
        pltpu.make_async_copy(
            x_hbm.at[pl.ds(xrow0 + step * TB, TB), :],
            xbuf.at[b],
            sems.at[b],
        ).wait()

    for b in range(NB):
        start_dma(b, b)

    nsteps = RPW // TB

    lane = lax.iota(jnp.int32, 16)

    def outer(i, carry):
        g0 = i * NB
        decv = jnp.zeros((16,), jnp.float32)
        for b in range(NB):
            g = g0 + b
            wait_dma(g, b)

            for sub in range(TB // TBC):

                def chunk(ci, accs, _b=b, _sub=sub):
                    base = ci * 16
                    wvs = [wt_v[j, pl.ds(base, 16)] for j in range(BITS)]
                    out = []
                    for t in range(TBC):
                        xv = xbuf[_b, _sub * TBC + t, pl.ds(base, 16)]
                        xr = _round_bf16(xv)
                        for j in range(BITS):
                            out.append(accs[t * BITS + j] + xr * wvs[j])
                    return tuple(out)

                accs = lax.fori_loop(
                    0, D // 16, chunk,
                    tuple(jnp.zeros((16,), jnp.float32)
                          for _ in range(TBC * BITS)))

                for t in range(TBC):
                    dec = jnp.zeros((16,), jnp.float32)
                    for j in range(BITS):
                        h = _hsum(accs[t * BITS + j], lane)
                        dec = dec + jnp.where(
                            h > 0, jnp.float32(1 << (BITS - 1 - j)),
                            jnp.float32(0))
                    decv = jnp.where(
                        lane == b * TB + sub * TBC + t, dec, decv)

            @pl.when(g + NB < nsteps)
            def _(_g=g, _b=b):
                start_dma(_g + NB, _b)

        decbuf[pl.ds(i * (NB * TB), NB * TB)] = decv
        return carry

    lax.fori_loop(0, nsteps // NB, outer, 0)

    pltpu.sync_copy(decbuf, dec_hbm.at[pl.ds(row0, RPW)])
    pltpu.sync_copy(gbuf, gates_hbm.at[pl.ds(row0, RPW)])


def _tc_part(x, W):
    return pl.pallas_call(
        _lsh_tc_kernel,
        grid=(M_TC // BM,),
        in_specs=[
            pl.BlockSpec((BM, D), lambda i: (i, 0)),
            pl.BlockSpec((D, BITS), lambda i: (0, 0)),
        ],
        out_specs=[
            pl.BlockSpec((BM,), lambda i: (i,)),
            pl.BlockSpec((BM,), lambda i: (i,)),
        ],
        out_shape=[
            jax.ShapeDtypeStruct((M_TC,), jnp.float32),
            jax.ShapeDtypeStruct((M_TC,), jnp.float32),
        ],
    )(x, W)


def _sc_part(x, wt):
    mesh = plsc.VectorSubcoreMesh(core_axis_name="c", subcore_axis_name="s")
    run = pl.kernel(
        _lsh_sc_body,
        out_type=[
            jax.ShapeDtypeStruct((M_SC,), jnp.float32),
            jax.ShapeDtypeStruct((M_SC,), jnp.float32),
        ],
        mesh=mesh,
        scratch_types=[
            pltpu.VMEM((BITS, D), jnp.float32),
            pltpu.VMEM((NB, TB, D), jnp.float32),
            pltpu.VMEM((RPW,), jnp.float32),
            pltpu.VMEM((RPW,), jnp.float32),
            pltpu.SemaphoreType.DMA((NB,)),
        ],
    )
    return run(x, wt)


def _sc_part_interp(x, wt):
    mesh = plsc.VectorSubcoreMesh(core_axis_name="c", subcore_axis_name="s")
    run = pl.kernel(
        _lsh_sc_body,
        out_type=[
            jax.ShapeDtypeStruct((M_SC,), jnp.float32),
            jax.ShapeDtypeStruct((M_SC,), jnp.float32),
        ],
        mesh=mesh,
        interpret=True,
        scratch_types=[
            pltpu.VMEM((BITS, D), jnp.float32),
            pltpu.VMEM((NB, TB, D), jnp.float32),
            pltpu.VMEM((RPW,), jnp.float32),
            pltpu.VMEM((RPW,), jnp.float32),
            pltpu.SemaphoreType.DMA((NB,)),
        ],
    )
    return run(x, wt)


def kernel(x, W):
    wt = W.T
    gates_tc, dec_tc = _tc_part(x, W)
    gates_sc, dec_sc = _sc_part(x, wt)
    gates = jnp.concatenate([gates_tc, gates_sc])
    dec = jnp.concatenate([dec_tc, dec_sc])
    return gates, dec


# speedup vs baseline: 1.4480x; 1.0086x over previous
import functools

import jax
import jax.numpy as jnp
from jax import lax
from jax.experimental import pallas as pl
from jax.experimental.pallas import tpu as pltpu
from jax.experimental.pallas import tpu_sc as plsc

N_TOK = 32768
D = 4096
BITS = 6

BM = 1024
M_SC = 1024
M_TC = N_TOK - M_SC

NW = 32
RPW = M_SC // NW
TB = 8
NB = 2
TBC = 4


def _lsh_tc_kernel(x_ref, w_ref, gates_ref, dec_ref):
    xb = x_ref[...].astype(jnp.bfloat16)
    wb = w_ref[...].astype(jnp.bfloat16)
    ht = jax.lax.dot_general(
        wb, xb, (((0,), (1,)), ((), ())),
        preferred_element_type=jnp.float32,
    )
    powers = (1 << jnp.arange(BITS - 1, -1, -1, dtype=jnp.int32)).astype(
        jnp.float32)[:, None]
    dec = jnp.sum(jnp.where(ht > 0, powers, 0.0), axis=0)
    dec_ref[...] = dec
    gates_ref[...] = jnp.ones_like(dec)


_GATHER_DNUMS = lax.GatherDimensionNumbers(
    offset_dims=(), collapsed_slice_dims=(0,), start_index_map=(0,))


def _shuffle(v, idx):
    return lax.gather(
        v, idx[:, None], _GATHER_DNUMS, slice_sizes=(1,),
        mode=lax.GatherScatterMode.PROMISE_IN_BOUNDS)


def _round_bf16(v):
    u = lax.bitcast_convert_type(v, jnp.uint32)
    rb = (u >> 16) & jnp.uint32(1)
    u = (u + jnp.uint32(0x7FFF) + rb) & jnp.uint32(0xFFFF0000)
    return lax.bitcast_convert_type(u, jnp.float32)


def _hsum(v, lane):
    for sh in (8, 4, 2, 1):
        v = v + _shuffle(v, lane ^ sh)
    return v


def _lsh_sc_body(x_hbm, wt_hbm, gates_hbm, dec_hbm,
                 wt_v, xbuf, decbuf, gbuf, sems):
    cid = lax.axis_index("c")
    sid = lax.axis_index("s")
    wid = sid * 2 + cid
    row0 = wid * RPW
    xrow0 = M_TC + row0

    pltpu.sync_copy(wt_hbm, wt_v)

    def round_w(ci, carry):
        base = ci * 16
        for j in range(BITS):
            wt_v[j, pl.ds(base, 16)] = _round_bf16(wt_v[j, pl.ds(base, 16)])
        return carry

    lax.fori_loop(0, D // 16, round_w, 0)

    ones16 = jnp.ones((16,), jnp.float32)
    for i in range(RPW // 16):
        gbuf[pl.ds(i * 16, 16)] = ones16

    def start_dma(step, b):
        pltpu.make_async_copy(
            x_hbm.at[pl.ds(xrow0 + step * TB, TB), :],
            xbuf.at[b],
            sems.at[b],
        ).start()

    def wait_dma(step, b):
        pltpu.make_async_copy(
            x_hbm.at[pl.ds(xrow0 + step * TB, TB), :],
            xbuf.at[b],
            sems.at[b],
        ).wait()

    for b in range(NB):
        start_dma(b, b)

    nsteps = RPW // TB

    lane = lax.iota(jnp.int32, 16)

    def outer(i, carry):
        g0 = i * NB
        decv = jnp.zeros((16,), jnp.float32)
        for b in range(NB):
            g = g0 + b
            wait_dma(g, b)

            for sub in range(TB // TBC):

                def chunk(ci, accs, _b=b, _sub=sub):
                    base = ci * 16
                    wvs = [wt_v[j, pl.ds(base, 16)] for j in range(BITS)]
                    out = []
                    for t in range(TBC):
                        xv = xbuf[_b, _sub * TBC + t, pl.ds(base, 16)]
                        xr = _round_bf16(xv)
                        for j in range(BITS):
                            out.append(accs[t * BITS + j] + xr * wvs[j])
                    return tuple(out)

                accs = lax.fori_loop(
                    0, D // 16, chunk,
                    tuple(jnp.zeros((16,), jnp.float32)
                          for _ in range(TBC * BITS)))

                for t in range(TBC):
                    dec = jnp.zeros((16,), jnp.float32)
                    for j in range(BITS):
                        h = _hsum(accs[t * BITS + j], lane)
                        dec = dec + jnp.where(
                            h > 0, jnp.float32(1 << (BITS - 1 - j)),
                            jnp.float32(0))
                    decv = jnp.where(
                        lane == b * TB + sub * TBC + t, dec, decv)

            @pl.when(g + NB < nsteps)
            def _(_g=g, _b=b):
                start_dma(_g + NB, _b)

        decbuf[pl.ds(i * (NB * TB), NB * TB)] = decv
        return carry

    lax.fori_loop(0, nsteps // NB, outer, 0)

    pltpu.sync_copy(decbuf, dec_hbm.at[pl.ds(row0, RPW)])
    pltpu.sync_copy(gbuf, gates_hbm.at[pl.ds(row0, RPW)])


def _tc_part(x, W):
    return pl.pallas_call(
        _lsh_tc_kernel,
        grid=(M_TC // BM,),
        in_specs=[
            pl.BlockSpec((BM, D), lambda i: (i, 0)),
            pl.BlockSpec((D, BITS), lambda i: (0, 0)),
        ],
        out_specs=[
            pl.BlockSpec((BM,), lambda i: (i,)),
            pl.BlockSpec((BM,), lambda i: (i,)),
        ],
        out_shape=[
            jax.ShapeDtypeStruct((M_TC,), jnp.float32),
            jax.ShapeDtypeStruct((M_TC,), jnp.float32),
        ],
    )(x, W)


def _sc_part(x, wt):
    mesh = plsc.VectorSubcoreMesh(core_axis_name="c", subcore_axis_name="s")
    run = pl.kernel(
        _lsh_sc_body,
        out_type=[
            jax.ShapeDtypeStruct((M_SC,), jnp.float32),
            jax.ShapeDtypeStruct((M_SC,), jnp.float32),
        ],
        mesh=mesh,
        scratch_types=[
            pltpu.VMEM((BITS, D), jnp.float32),
            pltpu.VMEM((NB, TB, D), jnp.float32),
            pltpu.VMEM((RPW,), jnp.float32),
            pltpu.VMEM((RPW,), jnp.float32),
            pltpu.SemaphoreType.DMA((NB,)),
        ],
    )
    return run(x, wt)


def _sc_part_interp(x, wt):
    mesh = plsc.VectorSubcoreMesh(core_axis_name="c", subcore_axis_name="s")
    run = pl.kernel(
        _lsh_sc_body,
        out_type=[
            jax.ShapeDtypeStruct((M_SC,), jnp.float32),
            jax.ShapeDtypeStruct((M_SC,), jnp.float32),
        ],
        mesh=mesh,
        interpret=True,
        scratch_types=[
            pltpu.VMEM((BITS, D), jnp.float32),
            pltpu.VMEM((NB, TB, D), jnp.float32),
            pltpu.VMEM((RPW,), jnp.float32),
            pltpu.VMEM((RPW,), jnp.float32),
            pltpu.SemaphoreType.DMA((NB,)),
        ],
    )
    return run(x, wt)


def kernel(x, W):
    wt = W.T
    gates_tc, dec_tc = _tc_part(x, W)
    gates_sc, dec_sc = _sc_part(x, wt)
    gates = jnp.concatenate([gates_tc, gates_sc])
    dec = jnp.concatenate([dec_tc, dec_sc])
    return gates, dec
